# baseline (speedup 1.0000x reference)
import functools

import jax
import jax.numpy as jnp
from jax import lax
from jax.experimental import pallas as pl
from jax.experimental.pallas import tpu as pltpu
from jax.experimental.pallas import tpu_sc as plsc

N_NODES = 10000
D = 256
DH = 128
N_SC = 2
N_TILES = 16
CHUNK = 128
ROWS_PER_TILE = 624
TAIL_ROWS = N_NODES - N_TILES * ROWS_PER_TILE
ACC_ROWS = 16
N_PHASES = 2


def _sc_aggregate(feat_cat, feat_full, src_lo, src_hi, dst_idx):
    n_chunks = src_lo.shape[1]
    ch_per_phase = n_chunks // N_PHASES
    mesh = plsc.VectorSubcoreMesh(core_axis_name="c", subcore_axis_name="s")

    @functools.partial(
        pl.kernel,
        mesh=mesh,
        out_type=jax.ShapeDtypeStruct((N_SC * N_NODES, DH), jnp.float32),
        scratch_types=[
            pltpu.VMEM_SHARED((ACC_ROWS, DH), jnp.float32),
            pltpu.VMEM((ch_per_phase, CHUNK), jnp.int32),
            pltpu.VMEM((ch_per_phase, CHUNK), jnp.int32),
            pltpu.VMEM((2, CHUNK, D), jnp.float32),
            pltpu.SemaphoreType.DMA,
            pltpu.SemaphoreType.DMA,
        ],
    )
    def agg(feat_hbm, feat_full_hbm, src_lo_hbm, src_hi_hbm, dst_hbm, out_hbm,
            acc, src_v, dst_v, rows_v, gsem, ssem):
        c = lax.axis_index("c")
        s = lax.axis_index("s")
        node0 = s * ROWS_PER_TILE

        plsc.subcore_barrier()

        def run_phase(phase):
            ch0 = phase * ch_per_phase

            @pl.when(c == 0)
            def _():
                pltpu.sync_copy(
                    src_lo_hbm.at[s, pl.ds(ch0, ch_per_phase)], src_v)

            @pl.when(c == 1)
            def _():
                pltpu.sync_copy(
                    src_hi_hbm.at[s, pl.ds(ch0, ch_per_phase)], src_v)

            pltpu.sync_copy(dst_hbm.at[s, pl.ds(ch0, ch_per_phase)], dst_v)

            def body(j, carry):
                b = j % 2

                @pl.when(j + 1 < ch_per_phase)
                def _():
                    pltpu.async_copy(
                        feat_full_hbm.at[src_v.at[j + 1]], rows_v.at[1 - b],
                        gsem)

                pltpu.make_async_copy(
                    feat_full_hbm.at[src_v.at[j]], rows_v.at[b], gsem).wait()
                return carry

            pltpu.async_copy(feat_full_hbm.at[src_v.at[0]], rows_v.at[0], gsem)
            lax.fori_loop(0, ch_per_phase, body, 0)

        for phase in range(N_PHASES):
            run_phase(phase)

        plsc.subcore_barrier()
        pltpu.sync_copy(
            rows_v.at[0, :, :DH],
            out_hbm.at[pl.ds(c * N_NODES + node0, CHUNK)])

    return agg(feat_cat, feat_full, src_lo, src_hi, dst_idx)


def _tc_linear(rst_cat, W, b2):
    MB = 1000
    nblk = N_NODES // MB

    def body(lo_ref, hi_ref, w_ref, b_ref, out_ref):
        w = w_ref[...]
        acc = lax.dot_general(lo_ref[...], w[:, :DH],
                              (((1,), (1,)), ((), ())),
                              preferred_element_type=jnp.float32)
        acc = acc + lax.dot_general(hi_ref[...], w[:, DH:],
                                    (((1,), (1,)), ((), ())),
                                    preferred_element_type=jnp.float32)
        out_ref[...] = acc + b_ref[...]

    return pl.pallas_call(
        body,
        grid=(nblk,),
        in_specs=[
            pl.BlockSpec((MB, DH), lambda i: (i, 0)),
            pl.BlockSpec((MB, DH), lambda i: (i + nblk, 0)),
            pl.BlockSpec((D, D), lambda i: (0, 0)),
            pl.BlockSpec((1, D), lambda i: (0, 0)),
        ],
        out_specs=pl.BlockSpec((MB, D), lambda i: (i, 0)),
        out_shape=jax.ShapeDtypeStruct((N_NODES, D), jnp.float32),
    )(rst_cat, rst_cat, W, b2)


def kernel(feat, edge_index, W, b):
    src = edge_index[0].astype(jnp.int32)
    dst = edge_index[1].astype(jnp.int32)
    e = src.shape[0]
    n_chunks = -(-e // (N_TILES * CHUNK))
    n_chunks = -(-n_chunks // (8 * N_PHASES)) * (8 * N_PHASES)
    e_pad = N_TILES * n_chunks * CHUNK
    pad = e_pad - e

    feat_cat = jnp.concatenate([feat[:, :DH], feat[:, DH:]], axis=0)

    src_p = jnp.concatenate([src, jnp.zeros((pad,), jnp.int32)])
    dst_p = jnp.concatenate([dst, jnp.full((pad,), N_NODES, jnp.int32)])
    src_lo = src_p.reshape(N_TILES, n_chunks, CHUNK)
    src_hi = src_lo
    dst_r = dst_p.reshape(N_TILES, n_chunks, CHUNK)

    rst_cat = _sc_aggregate(feat_cat, feat, src_lo, src_hi, dst_r)
    return _tc_linear(rst_cat, W, b.reshape(1, D))


# speedup vs baseline: 1.0492x; 1.0492x over previous
import functools

import numpy as np
import jax
import jax.numpy as jnp
from jax import lax
from jax.experimental import pallas as pl
from jax.experimental.pallas import tpu as pltpu
from jax.experimental.pallas import tpu_sc as plsc

N_NODES = 10000
D = 256
HALF_N = N_NODES // 2
N_SC = 2
N_TILES = 16
EPT = 10240
PIECE = 1024
N_PIECES = EPT // PIECE
CH = 64
ACC_ROWS = HALF_N + 8
CMP_LEN = EPT + 16 * 16
PACK_SHIFT = 14
TRASH = HALF_N
INIT_RPT = HALF_N // N_TILES // 8 * 8
INIT_TAIL = HALF_N - N_TILES * INIT_RPT

_k = np.arange(D)
_blk, _j = _k // 32, _k % 32
SIGMA = np.where(_j % 2 == 0, _blk * 32 + _j // 2, _blk * 32 + 16 + _j // 2)


def _sc_aggregate(feat, tbl, srcp, dstp):
    mesh = plsc.VectorSubcoreMesh(core_axis_name="c", subcore_axis_name="s")

    @functools.partial(
        pl.kernel,
        mesh=mesh,
        out_type=jax.ShapeDtypeStruct((2 * N_NODES, D // 2), jnp.float32),
        scratch_types=[
            pltpu.VMEM_SHARED((2 * ACC_ROWS, D // 2), jnp.float32),
            pltpu.VMEM((1, PIECE), jnp.int32),
            pltpu.VMEM((1, PIECE), jnp.int32),
            pltpu.VMEM((CMP_LEN,), jnp.int32),
            pltpu.VMEM((CH,), jnp.int32),
            pltpu.VMEM((2 * CH,), jnp.int32),
            pltpu.VMEM((CH,), jnp.int32),
            pltpu.VMEM((2 * CH,), jnp.int32),
            pltpu.VMEM((2, CH, D // 2), jnp.int32),
            pltpu.VMEM((2 * CH, D // 2), jnp.float32),
            pltpu.SemaphoreType.DMA,
        ],
    )
    def agg(feat_hbm, tbl_hbm, srcp_hbm, dstp_hbm, out_hbm,
            acc, raw_src, raw_dst, compact, src_i0, dst_i0, src_i1,
            dst_i1, rows, stage, gsem):
        c = lax.axis_index("c")
        s = lax.axis_index("s")
        lo = c * HALF_N
        a0 = s * INIT_RPT

        pltpu.sync_copy(feat_hbm.at[pl.ds(2 * (lo + a0), 2 * INIT_RPT)],
                        acc.at[pl.ds(2 * a0, 2 * INIT_RPT)])

        @pl.when(s == N_TILES - 1)
        def _():
            t0 = N_TILES * INIT_RPT
            pltpu.sync_copy(
                feat_hbm.at[pl.ds(2 * (lo + t0), 2 * INIT_TAIL)],
                acc.at[pl.ds(2 * t0, 2 * INIT_TAIL)])

        lane = lax.iota(jnp.int32, 16)
        target = lane + 1
        pib = "promise_in_bounds"
        tpk = TRASH << PACK_SHIFT

        def route_piece(p, carry):
            pltpu.sync_copy(srcp_hbm.at[s * N_PIECES + p], raw_src)
            pltpu.sync_copy(dstp_hbm.at[s * N_PIECES + p], raw_dst)

            def vec_body(v, carry):
                cnt16, np_, pend = carry
                sv = raw_src[0, pl.ds(v * 16, 16)]
                dv = raw_dst[0, pl.ds(v * 16, 16)]
                dl = dv - lo
                m = (dl >= 0) & (dl < HALF_N)
                packed = (dl << PACK_SHIFT) | sv
                x = jnp.where(m, 1, 0)
                pfx = x
                for sh in (1, 2, 4, 8):
                    shifted = pfx.at[jnp.maximum(lane - sh, 0)].get(mode=pib)
                    pfx = pfx + jnp.where(lane >= sh, shifted, 0)
                k = pfx[15]
                sel = jnp.zeros((16,), jnp.int32)
                for step in (8, 4, 2, 1):
                    cand = sel + step
                    pc = pfx.at[cand - 1].get(mode=pib)
                    sel = jnp.where(pc < target, cand, sel)
                kv = packed.at[jnp.minimum(sel, 15)].get(mode=pib)
                shift_in = kv.at[jnp.clip(lane - np_, 0, 15)].get(mode=pib)
                merged = jnp.where(lane < np_, pend, shift_in)
                compact[pl.ds(cnt16 * 16, 16)] = merged
                newtot = np_ + k
                emit = newtot >= 16
                cnt16 = cnt16 + jnp.where(emit, 1, 0)
                spill = kv.at[jnp.clip(lane + 16 - np_, 0, 15)].get(mode=pib)
                pend = jnp.where(emit, spill, merged)
                np_ = jnp.where(emit, newtot - 16, newtot)
                return cnt16, np_, pend

            return lax.fori_loop(0, PIECE // 16, vec_body, carry)

        cnt16, np_, pend = lax.fori_loop(
            0, N_PIECES, route_piece,
            (0, 0, jnp.zeros((16,), jnp.int32)))

        compact[pl.ds(cnt16 * 16, 16)] = jnp.where(lane < np_, pend, tpk)
        tvec = jnp.full((16,), tpk, jnp.int32)
        for j in range(1, 16):
            compact[pl.ds((cnt16 + j) * 16, 16)] = tvec
        cnt = cnt16 * 16 + np_

        plsc.subcore_barrier()

        n_pairs = jnp.maximum((cnt + 2 * CH - 1) // (2 * CH), 1)
        n_ch = 2 * n_pairs

        half = lane >> 1
        bit = lane & 1

        def prep(ch, sref, dref):
            base = ch * CH
            for k in range(CH // 16):
                pk = compact[pl.ds(base + k * 16, 16)]
                sref[pl.ds(k * 16, 16)] = pk & ((1 << PACK_SHIFT) - 1)
                d2 = (pk >> PACK_SHIFT) * 2
                da = d2.at[half].get(mode=pib) + bit
                db = d2.at[half + 8].get(mode=pib) + bit
                dref[pl.ds(k * 32, 16)] = da
                dref[pl.ds(k * 32 + 16, 16)] = db

        def finish(b, dref):
            pltpu.make_async_copy(tbl_hbm.at[src_i0], rows.at[b],
                                  gsem).wait()
            rv = rows.at[b]
            for r in range(CH):
                for k in range(D // 32):
                    w = rv[r, pl.ds(k * 16, 16)]
                    rr = 2 * r + (k // 4)
                    c0 = (32 * k) % 128
                    stage[rr, pl.ds(c0, 16)] = lax.bitcast_convert_type(
                        w << 16, jnp.float32)
                    stage[rr, pl.ds(c0 + 16, 16)] = lax.bitcast_convert_type(
                        w & jnp.int32(-65536), jnp.float32)
            pltpu.sync_copy(stage, acc.at[dref], add=True)

        prep(0, src_i0, dst_i0)
        pltpu.async_copy(tbl_hbm.at[src_i0], rows.at[0], gsem)

        def pair_body(mm, carry):
            ch0 = 2 * mm

            @pl.when(ch0 + 1 < n_ch)
            def _():
                prep(ch0 + 1, src_i1, dst_i1)
                pltpu.async_copy(tbl_hbm.at[src_i1], rows.at[1], gsem)

            @pl.when(ch0 < n_ch)
            def _():
                finish(0, dst_i0)

            @pl.when(ch0 + 2 < n_ch)
            def _():
                prep(ch0 + 2, src_i0, dst_i0)
                pltpu.async_copy(tbl_hbm.at[src_i0], rows.at[0], gsem)

            @pl.when(ch0 + 1 < n_ch)
            def _():
                finish(1, dst_i1)

            return carry

        lax.fori_loop(0, EPT // (2 * CH), pair_body, 0)

        plsc.subcore_barrier()
        pltpu.sync_copy(acc.at[pl.ds(2 * a0, 2 * INIT_RPT)],
                        out_hbm.at[pl.ds(2 * (lo + a0), 2 * INIT_RPT)])

        @pl.when(s == N_TILES - 1)
        def _():
            t0 = N_TILES * INIT_RPT
            pltpu.sync_copy(acc.at[pl.ds(2 * t0, 2 * INIT_TAIL)],
                            out_hbm.at[pl.ds(2 * (lo + t0), 2 * INIT_TAIL)])

    return agg(feat, tbl, srcp, dstp)


def _tc_linear(rst, W, b2):
    MB = 1000

    def body(x_ref, w_ref, b_ref, o_ref):
        o_ref[...] = lax.dot_general(
            x_ref[...], w_ref[...], (((1,), (1,)), ((), ())),
            preferred_element_type=jnp.float32) + b_ref[...]

    return pl.pallas_call(
        body,
        grid=(N_NODES // MB,),
        in_specs=[
            pl.BlockSpec((MB, D), lambda i: (i, 0)),
            pl.BlockSpec((D, D), lambda i: (0, 0)),
            pl.BlockSpec((1, D), lambda i: (0, 0)),
        ],
        out_specs=pl.BlockSpec((MB, D), lambda i: (i, 0)),
        out_shape=jax.ShapeDtypeStruct((N_NODES, D), jnp.float32),
    )(rst, W, b2)


def kernel(feat, edge_index, W, b):
    src = edge_index[0].astype(jnp.int32)
    dst = edge_index[1].astype(jnp.int32)
    e = src.shape[0]
    e_pad = N_TILES * EPT
    pad = e_pad - e

    srcp = jnp.concatenate([src, jnp.zeros((pad,), jnp.int32)]).reshape(
        N_TILES * N_PIECES, 1, PIECE)
    dstp = jnp.concatenate([dst, jnp.full((pad,), 1 << 20, jnp.int32)]
                           ).reshape(N_TILES * N_PIECES, 1, PIECE)

    tbl = jax.lax.bitcast_convert_type(
        feat[:, SIGMA].astype(jnp.bfloat16).reshape(N_NODES, D // 2, 2),
        jnp.int32)

    rst2 = _sc_aggregate(feat.reshape(2 * N_NODES, D // 2), tbl, srcp, dstp)
    return _tc_linear(rst2.reshape(N_NODES, D), W, b.reshape(1, D))


# speedup vs baseline: 1.4723x; 1.4032x over previous
import functools

import jax
import jax.numpy as jnp
from jax import lax
from jax.experimental import pallas as pl
from jax.experimental.pallas import tpu as pltpu
from jax.experimental.pallas import tpu_sc as plsc

N_NODES = 10000
D = 256
DH = 128
N_SC = 2
N_TILES = 16
CHUNK = 128
ROWS_PER_TILE = 624
TAIL_ROWS = N_NODES - N_TILES * ROWS_PER_TILE
ACC_ROWS = N_NODES + 8


def _sc_aggregate(feat_cat, src_lo, src_hi, dst_idx):
    n_chunks = src_lo.shape[1]
    mesh = plsc.VectorSubcoreMesh(core_axis_name="c", subcore_axis_name="s")

    @functools.partial(
        pl.kernel,
        mesh=mesh,
        out_type=jax.ShapeDtypeStruct((N_SC * N_NODES, DH), jnp.float32),
        scratch_types=[
            pltpu.VMEM_SHARED((ACC_ROWS, DH), jnp.float32),
            pltpu.VMEM((n_chunks, CHUNK), jnp.int32),
            pltpu.VMEM((n_chunks, CHUNK), jnp.int32),
            pltpu.VMEM((CHUNK, DH), jnp.float32),
            pltpu.SemaphoreType.DMA,
        ],
    )
    def agg(feat_hbm, src_lo_hbm, src_hi_hbm, dst_hbm, out_hbm,
            acc, src_v, dst_v, rows_v, sem):
        c = lax.axis_index("c")
        s = lax.axis_index("s")
        node0 = s * ROWS_PER_TILE

        pltpu.sync_copy(
            feat_hbm.at[pl.ds(c * N_NODES + node0, ROWS_PER_TILE)],
            acc.at[pl.ds(node0, ROWS_PER_TILE)])

        @pl.when(s == N_TILES - 1)
        def _():
            tail0 = N_TILES * ROWS_PER_TILE
            pltpu.sync_copy(
                feat_hbm.at[pl.ds(c * N_NODES + tail0, TAIL_ROWS)],
                acc.at[pl.ds(tail0, TAIL_ROWS)])

        @pl.when(c == 0)
        def _():
            pltpu.sync_copy(src_lo_hbm.at[s], src_v)

        @pl.when(c == 1)
        def _():
            pltpu.sync_copy(src_hi_hbm.at[s], src_v)

        pltpu.sync_copy(dst_hbm.at[s], dst_v)
        plsc.subcore_barrier()

        def body(j, carry):
            pltpu.async_copy(feat_hbm.at[src_v.at[j]], rows_v, sem).wait()
            pltpu.sync_copy(rows_v, acc.at[dst_v.at[j]], add=True)
            return carry

        lax.fori_loop(0, n_chunks, body, 0)

        plsc.subcore_barrier()
        pltpu.sync_copy(
            acc.at[pl.ds(node0, ROWS_PER_TILE)],
            out_hbm.at[pl.ds(c * N_NODES + node0, ROWS_PER_TILE)])

        @pl.when(s == N_TILES - 1)
        def _():
            tail0 = N_TILES * ROWS_PER_TILE
            pltpu.sync_copy(
                acc.at[pl.ds(tail0, TAIL_ROWS)],
                out_hbm.at[pl.ds(c * N_NODES + tail0, TAIL_ROWS)])

    return agg(feat_cat, src_lo, src_hi, dst_idx)


def _tc_linear(rst_cat, W, b2):
    MB = 1000
    nblk = N_NODES // MB

    def body(lo_ref, hi_ref, w_ref, b_ref, out_ref):
        w = w_ref[...]
        acc = lax.dot_general(lo_ref[...], w[:, :DH],
                              (((1,), (1,)), ((), ())),
                              preferred_element_type=jnp.float32)
        acc = acc + lax.dot_general(hi_ref[...], w[:, DH:],
                                    (((1,), (1,)), ((), ())),
                                    preferred_element_type=jnp.float32)
        out_ref[...] = acc + b_ref[...]

    return pl.pallas_call(
        body,
        grid=(nblk,),
        in_specs=[
            pl.BlockSpec((MB, DH), lambda i: (i, 0)),
            pl.BlockSpec((MB, DH), lambda i: (i + nblk, 0)),
            pl.BlockSpec((D, D), lambda i: (0, 0)),
            pl.BlockSpec((1, D), lambda i: (0, 0)),
        ],
        out_specs=pl.BlockSpec((MB, D), lambda i: (i, 0)),
        out_shape=jax.ShapeDtypeStruct((N_NODES, D), jnp.float32),
    )(rst_cat, rst_cat, W, b2)


def kernel(feat, edge_index, W, b):
    src = edge_index[0].astype(jnp.int32)
    dst = edge_index[1].astype(jnp.int32)
    e = src.shape[0]
    n_chunks = -(-e // (N_TILES * CHUNK))
    e_pad = N_TILES * n_chunks * CHUNK
    pad = e_pad - e

    feat_cat = jnp.concatenate([feat[:, :DH], feat[:, DH:]], axis=0)

    src_p = jnp.concatenate([src, jnp.zeros((pad,), jnp.int32)])
    dst_p = jnp.concatenate([dst, jnp.full((pad,), N_NODES, jnp.int32)])
    src_lo = src_p.reshape(N_TILES, n_chunks, CHUNK)
    src_hi = src_lo + N_NODES
    dst_r = dst_p.reshape(N_TILES, n_chunks, CHUNK)

    rst_cat = _sc_aggregate(feat_cat, src_lo, src_hi, dst_r)
    return _tc_linear(rst_cat, W, b.reshape(1, D))
